# baseline (speedup 1.0000x reference)
import functools

import jax
import jax.numpy as jnp
from jax import lax
from jax.experimental import pallas as pl
from jax.experimental.pallas import tpu as pltpu
from jax.experimental.pallas import tpu_sc as plsc

N = 10000
F = 128
E = 320000

NC = 2
NS = 16
NW = NC * NS

CHUNK = 128
NBUF = 2
IDEP = 8
J = 80
E_PAD = NW * CHUNK * J
NPAD = 10240
ROWS_PER_TILE = NPAD // NS
ZCH = 128

_mesh = plsc.VectorSubcoreMesh(core_axis_name="c", subcore_axis_name="s")


@jax.jit
def _sc_aggregate(z, src3, dst3, zeros):
    @functools.partial(
        pl.kernel,
        out_type=jax.ShapeDtypeStruct((NC, NPAD, F), jnp.float32),
        mesh=_mesh,
        scratch_types=[
            pltpu.VMEM((IDEP, CHUNK), jnp.int32),
            pltpu.VMEM((IDEP, CHUNK), jnp.int32),
            pltpu.VMEM((NBUF, CHUNK, F), jnp.float32),
            pltpu.VMEM_SHARED((NPAD, F), jnp.float32),
        ] + [pltpu.SemaphoreType.DMA] * (NBUF + 2 * IDEP),
    )
    def agg_kernel(z_hbm, src_hbm, dst_hbm, zeros_hbm, out_hbm,
                   src_r, dst_r, rows_v, agg_sh, *sems):
        cid = lax.axis_index("c")
        sid = lax.axis_index("s")
        wid = cid * NS + sid
        gsem = sems[:NBUF]
        isem_s = sems[NBUF:NBUF + IDEP]
        isem_d = sems[NBUF + IDEP:]

        def pref_idx(q, j):
            pltpu.async_copy(src_hbm.at[wid, j], src_r.at[q], isem_s[q])
            pltpu.async_copy(dst_hbm.at[wid, j], dst_r.at[q], isem_d[q])

        def gather(b, q, j):
            pltpu.make_async_copy(
                src_hbm.at[wid, j], src_r.at[q], isem_s[q]).wait()
            pltpu.async_copy(z_hbm.at[src_r.at[q]], rows_v.at[b], gsem[b])

        def scatter(b, q, j):
            pltpu.make_async_copy(
                z_hbm.at[src_r.at[q]], rows_v.at[b], gsem[b]).wait()
            pltpu.make_async_copy(
                dst_hbm.at[wid, j], dst_r.at[q], isem_d[q]).wait()
            pltpu.sync_copy(rows_v.at[b], agg_sh.at[dst_r.at[q]], add=True)

        for q in range(IDEP):
            pref_idx(q, q)

        row0 = sid * ROWS_PER_TILE
        @pl.loop(0, ROWS_PER_TILE, step=ZCH)
        def _(r):
            pltpu.sync_copy(zeros_hbm, agg_sh.at[pl.ds(row0 + r, ZCH)])

        plsc.subcore_barrier()

        for b in range(NBUF):
            gather(b, b, b)

        @pl.loop(0, J - IDEP, step=IDEP)
        def _(j):
            for b in range(IDEP):
                jb = j + b
                scatter(b % NBUF, b, jb)
                pref_idx(b, jb + IDEP)
                gather(b % NBUF, (b + NBUF) % IDEP, jb + NBUF)

        for b in range(IDEP):
            jb = J - IDEP + b
            scatter(b % NBUF, b, jb)
            if b < IDEP - NBUF:
                gather(b % NBUF, (b + NBUF) % IDEP, jb + NBUF)

        plsc.subcore_barrier()

        @pl.loop(0, ROWS_PER_TILE, step=ZCH)
        def _(r):
            pltpu.sync_copy(agg_sh.at[pl.ds(row0 + r, ZCH)],
                            out_hbm.at[cid, pl.ds(row0 + r, ZCH)])

    return agg_kernel(z, src3, dst3, zeros)


FD = 128

@jax.jit
def _sc_degree(dst3, zeros16, ones16):
    @functools.partial(
        pl.kernel,
        out_type=jax.ShapeDtypeStruct((NC, NPAD, FD), jnp.float32),
        mesh=_mesh,
        scratch_types=[
            pltpu.VMEM((IDEP, CHUNK), jnp.int32),
            pltpu.VMEM((CHUNK, FD), jnp.float32),
            pltpu.VMEM_SHARED((NPAD, FD), jnp.float32),
        ] + [pltpu.SemaphoreType.DMA] * IDEP,
    )
    def deg_kernel(dst_hbm, zeros_hbm, ones_hbm, out_hbm,
                   dst_r, ones_v, deg_sh, *isem):
        cid = lax.axis_index("c")
        sid = lax.axis_index("s")
        wid = cid * NS + sid

        for q in range(IDEP):
            pltpu.async_copy(dst_hbm.at[wid, q], dst_r.at[q], isem[q])
        pltpu.sync_copy(ones_hbm, ones_v)

        row0 = sid * ROWS_PER_TILE
        @pl.loop(0, ROWS_PER_TILE, step=ZCH)
        def _(r):
            pltpu.sync_copy(zeros_hbm, deg_sh.at[pl.ds(row0 + r, ZCH)])

        plsc.subcore_barrier()

        @pl.loop(0, J - IDEP, step=IDEP)
        def _(j):
            for b in range(IDEP):
                pltpu.make_async_copy(
                    dst_hbm.at[wid, j + b], dst_r.at[b], isem[b]).wait()
                pltpu.sync_copy(ones_v, deg_sh.at[dst_r.at[b]], add=True)
                pltpu.async_copy(
                    dst_hbm.at[wid, j + b + IDEP], dst_r.at[b], isem[b])

        for b in range(IDEP):
            jb = J - IDEP + b
            pltpu.make_async_copy(
                dst_hbm.at[wid, jb], dst_r.at[b], isem[b]).wait()
            pltpu.sync_copy(ones_v, deg_sh.at[dst_r.at[b]], add=True)

        plsc.subcore_barrier()

        @pl.loop(0, ROWS_PER_TILE, step=ZCH)
        def _(r):
            pltpu.sync_copy(deg_sh.at[pl.ds(row0 + r, ZCH)],
                            out_hbm.at[cid, pl.ds(row0 + r, ZCH)])

    return deg_kernel(dst3, zeros16, ones16)


BN = 400
NBLK = N // BN

_row = pl.BlockSpec((BN, F), lambda i: (i, 0))
_prt = pl.BlockSpec((NC, BN, F), lambda i: (0, i, 0))
_dis = pl.BlockSpec((BN, 1), lambda i: (i, 0))
_wsp = pl.BlockSpec((F, F), lambda i: (0, 0))
_vec = pl.BlockSpec((1, F), lambda i: (0, 0))

_rowF = jax.ShapeDtypeStruct((N, F), jnp.float32)
_disF = jax.ShapeDtypeStruct((NPAD, 1), jnp.float32)


def _tc_prep(pdeg, x):
    def body(p_ref, x_ref, dis_ref, z_ref):
        deg = p_ref[0, :, 0] + p_ref[1, :, 0]
        dis = jnp.where(deg > 0.0, lax.rsqrt(jnp.maximum(deg, 1.0)), 0.0)
        dis_ref[...] = dis[:, None]
        z_ref[...] = x_ref[...] * dis[:N, None]

    return pl.pallas_call(
        body,
        grid=(1,),
        in_specs=[pl.BlockSpec((NC, NPAD, FD), lambda i: (0, 0, 0)),
                  pl.BlockSpec((N, F), lambda i: (0, 0))],
        out_specs=[pl.BlockSpec((NPAD, 1), lambda i: (0, 0)),
                   pl.BlockSpec((N, F), lambda i: (0, 0))],
        out_shape=[_disF, _rowF],
    )(pdeg, x)


def _tc_step1(P, dis, x, w0, w1):
    def body(p_ref, dis_ref, x_ref, w0_ref, w1_ref, out_ref, t_ref, z_ref):
        d = dis_ref[...]
        s = -(p_ref[0] + p_ref[1]) * d
        t_ref[...] = s
        z_ref[...] = s * d
        out_ref[...] = (
            jnp.dot(x_ref[...], w0_ref[...], preferred_element_type=jnp.float32)
            + jnp.dot(s, w1_ref[...], preferred_element_type=jnp.float32))

    return pl.pallas_call(
        body,
        grid=(NBLK,),
        in_specs=[_prt, _dis, _row, _wsp, _wsp],
        out_specs=[_row, _row, _row],
        out_shape=[_rowF, _rowF, _rowF],
    )(P, dis, x, w0, w1)


def _tc_step_mid(P, dis, tpp, out_in, wk):
    def body(p_ref, dis_ref, tpp_ref, o_ref, wk_ref, out_ref, t_ref, z_ref):
        d = dis_ref[...]
        t = -2.0 * (p_ref[0] + p_ref[1]) * d - tpp_ref[...]
        t_ref[...] = t
        z_ref[...] = t * d
        out_ref[...] = o_ref[...] + jnp.dot(
            t, wk_ref[...], preferred_element_type=jnp.float32)

    return pl.pallas_call(
        body,
        grid=(NBLK,),
        in_specs=[_prt, _dis, _row, _row, _wsp],
        out_specs=[_row, _row, _row],
        out_shape=[_rowF, _rowF, _rowF],
    )(P, dis, tpp, out_in, wk)


def _tc_step_final(P, dis, tpp, out_in, wk, b, gamma, beta, res=None):
    last = res is not None

    def body(*refs):
        if last:
            (p_ref, dis_ref, tpp_ref, o_ref, wk_ref, b_ref, g_ref, be_ref,
             r_ref, y_ref) = refs
        else:
            (p_ref, dis_ref, tpp_ref, o_ref, wk_ref, b_ref, g_ref, be_ref,
             xn_ref, z_ref) = refs
        d = dis_ref[...]
        t = -2.0 * (p_ref[0] + p_ref[1]) * d - tpp_ref[...]
        o = o_ref[...] + jnp.dot(
            t, wk_ref[...], preferred_element_type=jnp.float32) + b_ref[...]
        h = jnp.maximum(o, 0.0)
        mu = jnp.mean(h, axis=-1, keepdims=True)
        var = jnp.mean((h - mu) ** 2, axis=-1, keepdims=True)
        hn = (h - mu) / jnp.sqrt(var + 1e-6) * g_ref[...] + be_ref[...]
        if last:
            y_ref[...] = hn + r_ref[...]
        else:
            xn_ref[...] = hn
            z_ref[...] = hn * d

    in_specs = [_prt, _dis, _row, _row, _wsp, _vec, _vec, _vec]
    args = [P, dis, tpp, out_in, wk, b, gamma, beta]
    if last:
        in_specs.append(_row)
        args.append(res)
        out_specs, out_shape = [_row], [_rowF]
    else:
        out_specs, out_shape = [_row, _row], [_rowF, _rowF]

    result = pl.pallas_call(
        body,
        grid=(NBLK,),
        in_specs=in_specs,
        out_specs=out_specs,
        out_shape=out_shape,
    )(*args)
    return result[0] if last else result


def _cheb_layer(x, z, dis, W, b, gamma, beta, src3, dst3, zeros, res=None):
    K = W.shape[0]
    P = _sc_aggregate(z, src3, dst3, zeros)
    out, t1, z = _tc_step1(P, dis, x, W[0], W[1])
    tprev2, tprev1 = x, t1
    for k in range(2, K - 1):
        P = _sc_aggregate(z, src3, dst3, zeros)
        out, tk, z = _tc_step_mid(P, dis, tprev2, out, W[k])
        tprev2, tprev1 = tprev1, tk
    P = _sc_aggregate(z, src3, dst3, zeros)
    b2 = b.reshape(1, F)
    g2 = gamma.reshape(1, F)
    be2 = beta.reshape(1, F)
    return _tc_step_final(P, dis, tprev2, out, W[K - 1], b2, g2, be2, res=res)


def kernel(x, edge_index, W1, b1, gamma1, beta1, W2, b2, gamma2, beta2):
    src = edge_index[0]
    dst = edge_index[1]
    pad = E_PAD - E
    pad_ar = jnp.arange(pad, dtype=jnp.int32)
    src_p = jnp.concatenate([src, pad_ar % N])
    dst_p = jnp.concatenate([dst, N + pad_ar % (NPAD - N)])
    src3 = src_p.reshape(NW, J, CHUNK)
    dst3 = dst_p.reshape(NW, J, CHUNK)
    zeros = jnp.zeros((ZCH, F), jnp.float32)

    zeros16 = jnp.zeros((ZCH, FD), jnp.float32)
    ones16 = jnp.ones((CHUNK, FD), jnp.float32)
    pdeg = _sc_degree(dst3, zeros16, ones16)
    dis, z0 = _tc_prep(pdeg, x)

    x2, z2 = _cheb_layer(x, z0, dis, W1, b1, gamma1, beta1, src3, dst3, zeros)
    y = _cheb_layer(x2, z2, dis, W2, b2, gamma2, beta2, src3, dst3, zeros,
                    res=x)
    return y


# speedup vs baseline: 1.0005x; 1.0005x over previous
import functools

import jax
import jax.numpy as jnp
from jax import lax
from jax.experimental import pallas as pl
from jax.experimental.pallas import tpu as pltpu
from jax.experimental.pallas import tpu_sc as plsc

N = 10000
F = 128
E = 320000

NC = 2
NS = 16
NW = NC * NS

CHUNK = 128
NBUF = 2
IDEP = 4
J = 80
E_PAD = NW * CHUNK * J
NPAD = 10240
ROWS_PER_TILE = NPAD // NS
ZCH = 128

_mesh = plsc.VectorSubcoreMesh(core_axis_name="c", subcore_axis_name="s")


@jax.jit
def _sc_aggregate(z, src3, dst3, zeros):
    @functools.partial(
        pl.kernel,
        out_type=jax.ShapeDtypeStruct((NC, NPAD, F), jnp.float32),
        mesh=_mesh,
        scratch_types=[
            pltpu.VMEM((IDEP, CHUNK), jnp.int32),
            pltpu.VMEM((IDEP, CHUNK), jnp.int32),
            pltpu.VMEM((NBUF, CHUNK, F), jnp.float32),
            pltpu.VMEM_SHARED((NPAD, F), jnp.float32),
        ] + [pltpu.SemaphoreType.DMA] * (NBUF + 2 * IDEP),
    )
    def agg_kernel(z_hbm, src_hbm, dst_hbm, zeros_hbm, out_hbm,
                   src_r, dst_r, rows_v, agg_sh, *sems):
        cid = lax.axis_index("c")
        sid = lax.axis_index("s")
        wid = cid * NS + sid
        gsem = sems[:NBUF]
        isem_s = sems[NBUF:NBUF + IDEP]
        isem_d = sems[NBUF + IDEP:]

        def pref_idx(q, j):
            pltpu.async_copy(src_hbm.at[wid, j], src_r.at[q], isem_s[q])
            pltpu.async_copy(dst_hbm.at[wid, j], dst_r.at[q], isem_d[q])

        def gather(b, q, j):
            pltpu.make_async_copy(
                src_hbm.at[wid, j], src_r.at[q], isem_s[q]).wait()
            pltpu.async_copy(z_hbm.at[src_r.at[q]], rows_v.at[b], gsem[b])

        def scatter(b, q, j):
            pltpu.make_async_copy(
                z_hbm.at[src_r.at[q]], rows_v.at[b], gsem[b]).wait()
            pltpu.make_async_copy(
                dst_hbm.at[wid, j], dst_r.at[q], isem_d[q]).wait()
            pltpu.sync_copy(rows_v.at[b], agg_sh.at[dst_r.at[q]], add=True)

        for q in range(IDEP):
            pref_idx(q, q)

        row0 = sid * ROWS_PER_TILE
        @pl.loop(0, ROWS_PER_TILE, step=ZCH)
        def _(r):
            pltpu.sync_copy(zeros_hbm, agg_sh.at[pl.ds(row0 + r, ZCH)])

        plsc.subcore_barrier()

        for b in range(NBUF):
            gather(b, b, b)

        @pl.loop(0, J - IDEP, step=IDEP)
        def _(j):
            for b in range(IDEP):
                jb = j + b
                scatter(b % NBUF, b, jb)
                pref_idx(b, jb + IDEP)
                gather(b % NBUF, (b + NBUF) % IDEP, jb + NBUF)

        for b in range(IDEP):
            jb = J - IDEP + b
            scatter(b % NBUF, b, jb)
            if b < IDEP - NBUF:
                gather(b % NBUF, (b + NBUF) % IDEP, jb + NBUF)

        plsc.subcore_barrier()

        @pl.loop(0, ROWS_PER_TILE, step=ZCH)
        def _(r):
            pltpu.sync_copy(agg_sh.at[pl.ds(row0 + r, ZCH)],
                            out_hbm.at[cid, pl.ds(row0 + r, ZCH)])

    return agg_kernel(z, src3, dst3, zeros)


FD = 128

@jax.jit
def _sc_degree(dst3, zeros16, ones16):
    @functools.partial(
        pl.kernel,
        out_type=jax.ShapeDtypeStruct((NC, NPAD, FD), jnp.float32),
        mesh=_mesh,
        scratch_types=[
            pltpu.VMEM((IDEP, CHUNK), jnp.int32),
            pltpu.VMEM((CHUNK, FD), jnp.float32),
            pltpu.VMEM_SHARED((NPAD, FD), jnp.float32),
        ] + [pltpu.SemaphoreType.DMA] * IDEP,
    )
    def deg_kernel(dst_hbm, zeros_hbm, ones_hbm, out_hbm,
                   dst_r, ones_v, deg_sh, *isem):
        cid = lax.axis_index("c")
        sid = lax.axis_index("s")
        wid = cid * NS + sid

        for q in range(IDEP):
            pltpu.async_copy(dst_hbm.at[wid, q], dst_r.at[q], isem[q])
        pltpu.sync_copy(ones_hbm, ones_v)

        row0 = sid * ROWS_PER_TILE
        @pl.loop(0, ROWS_PER_TILE, step=ZCH)
        def _(r):
            pltpu.sync_copy(zeros_hbm, deg_sh.at[pl.ds(row0 + r, ZCH)])

        plsc.subcore_barrier()

        @pl.loop(0, J - IDEP, step=IDEP)
        def _(j):
            for b in range(IDEP):
                pltpu.make_async_copy(
                    dst_hbm.at[wid, j + b], dst_r.at[b], isem[b]).wait()
                pltpu.sync_copy(ones_v, deg_sh.at[dst_r.at[b]], add=True)
                pltpu.async_copy(
                    dst_hbm.at[wid, j + b + IDEP], dst_r.at[b], isem[b])

        for b in range(IDEP):
            jb = J - IDEP + b
            pltpu.make_async_copy(
                dst_hbm.at[wid, jb], dst_r.at[b], isem[b]).wait()
            pltpu.sync_copy(ones_v, deg_sh.at[dst_r.at[b]], add=True)

        plsc.subcore_barrier()

        @pl.loop(0, ROWS_PER_TILE, step=ZCH)
        def _(r):
            pltpu.sync_copy(deg_sh.at[pl.ds(row0 + r, ZCH)],
                            out_hbm.at[cid, pl.ds(row0 + r, ZCH)])

    return deg_kernel(dst3, zeros16, ones16)


BN = 400
NBLK = N // BN

_row = pl.BlockSpec((BN, F), lambda i: (i, 0))
_prt = pl.BlockSpec((NC, BN, F), lambda i: (0, i, 0))
_dis = pl.BlockSpec((BN, 1), lambda i: (i, 0))
_wsp = pl.BlockSpec((F, F), lambda i: (0, 0))
_vec = pl.BlockSpec((1, F), lambda i: (0, 0))

_rowF = jax.ShapeDtypeStruct((N, F), jnp.float32)
_disF = jax.ShapeDtypeStruct((NPAD, 1), jnp.float32)


def _tc_prep(pdeg, x):
    def body(p_ref, x_ref, dis_ref, z_ref):
        deg = p_ref[0, :, 0] + p_ref[1, :, 0]
        dis = jnp.where(deg > 0.0, lax.rsqrt(jnp.maximum(deg, 1.0)), 0.0)
        dis_ref[...] = dis[:, None]
        z_ref[...] = x_ref[...] * dis[:N, None]

    return pl.pallas_call(
        body,
        grid=(1,),
        in_specs=[pl.BlockSpec((NC, NPAD, FD), lambda i: (0, 0, 0)),
                  pl.BlockSpec((N, F), lambda i: (0, 0))],
        out_specs=[pl.BlockSpec((NPAD, 1), lambda i: (0, 0)),
                   pl.BlockSpec((N, F), lambda i: (0, 0))],
        out_shape=[_disF, _rowF],
    )(pdeg, x)


def _tc_recur1(P, dis):
    def body(p_ref, dis_ref, t_ref, z_ref):
        d = dis_ref[...]
        s = -(p_ref[0] + p_ref[1]) * d
        t_ref[...] = s
        z_ref[...] = s * d

    return pl.pallas_call(
        body,
        grid=(NBLK,),
        in_specs=[_prt, _dis],
        out_specs=[_row, _row],
        out_shape=[_rowF, _rowF],
    )(P, dis)


def _tc_recur(P, dis, tpp):
    def body(p_ref, dis_ref, tpp_ref, t_ref, z_ref):
        d = dis_ref[...]
        t = -2.0 * (p_ref[0] + p_ref[1]) * d - tpp_ref[...]
        t_ref[...] = t
        z_ref[...] = t * d

    return pl.pallas_call(
        body,
        grid=(NBLK,),
        in_specs=[_prt, _dis, _row],
        out_specs=[_row, _row],
        out_shape=[_rowF, _rowF],
    )(P, dis, tpp)


def _tc_mat2(x, w0, t1, w1):
    def body(x_ref, w0_ref, t_ref, w1_ref, out_ref):
        out_ref[...] = (
            jnp.dot(x_ref[...], w0_ref[...], preferred_element_type=jnp.float32)
            + jnp.dot(t_ref[...], w1_ref[...], preferred_element_type=jnp.float32))

    return pl.pallas_call(
        body,
        grid=(NBLK,),
        in_specs=[_row, _wsp, _row, _wsp],
        out_specs=_row,
        out_shape=_rowF,
    )(x, w0, t1, w1)


def _tc_matacc(out_in, t, wk):
    def body(o_ref, t_ref, wk_ref, out_ref):
        out_ref[...] = o_ref[...] + jnp.dot(
            t_ref[...], wk_ref[...], preferred_element_type=jnp.float32)

    return pl.pallas_call(
        body,
        grid=(NBLK,),
        in_specs=[_row, _row, _wsp],
        out_specs=_row,
        out_shape=_rowF,
    )(out_in, t, wk)


def _tc_step_final(P, dis, tpp, out_in, wk, b, gamma, beta, res=None):
    last = res is not None

    def body(*refs):
        if last:
            (p_ref, dis_ref, tpp_ref, o_ref, wk_ref, b_ref, g_ref, be_ref,
             r_ref, y_ref) = refs
        else:
            (p_ref, dis_ref, tpp_ref, o_ref, wk_ref, b_ref, g_ref, be_ref,
             xn_ref, z_ref) = refs
        d = dis_ref[...]
        t = -2.0 * (p_ref[0] + p_ref[1]) * d - tpp_ref[...]
        o = o_ref[...] + jnp.dot(
            t, wk_ref[...], preferred_element_type=jnp.float32) + b_ref[...]
        h = jnp.maximum(o, 0.0)
        mu = jnp.mean(h, axis=-1, keepdims=True)
        var = jnp.mean((h - mu) ** 2, axis=-1, keepdims=True)
        hn = (h - mu) / jnp.sqrt(var + 1e-6) * g_ref[...] + be_ref[...]
        if last:
            y_ref[...] = hn + r_ref[...]
        else:
            xn_ref[...] = hn
            z_ref[...] = hn * d

    in_specs = [_prt, _dis, _row, _row, _wsp, _vec, _vec, _vec]
    args = [P, dis, tpp, out_in, wk, b, gamma, beta]
    if last:
        in_specs.append(_row)
        args.append(res)
        out_specs, out_shape = [_row], [_rowF]
    else:
        out_specs, out_shape = [_row, _row], [_rowF, _rowF]

    result = pl.pallas_call(
        body,
        grid=(NBLK,),
        in_specs=in_specs,
        out_specs=out_specs,
        out_shape=out_shape,
    )(*args)
    return result[0] if last else result


def _cheb_layer(x, z, dis, W, b, gamma, beta, src3, dst3, zeros, res=None):
    K = W.shape[0]
    P = _sc_aggregate(z, src3, dst3, zeros)
    t1, z = _tc_recur1(P, dis)
    out = _tc_mat2(x, W[0], t1, W[1])
    tprev2, tprev1 = x, t1
    for k in range(2, K - 1):
        P = _sc_aggregate(z, src3, dst3, zeros)
        tk, z = _tc_recur(P, dis, tprev2)
        out = _tc_matacc(out, tk, W[k])
        tprev2, tprev1 = tprev1, tk
    P = _sc_aggregate(z, src3, dst3, zeros)
    b2 = b.reshape(1, F)
    g2 = gamma.reshape(1, F)
    be2 = beta.reshape(1, F)
    return _tc_step_final(P, dis, tprev2, out, W[K - 1], b2, g2, be2, res=res)


def kernel(x, edge_index, W1, b1, gamma1, beta1, W2, b2, gamma2, beta2):
    src = edge_index[0]
    dst = edge_index[1]
    pad = E_PAD - E
    pad_ar = jnp.arange(pad, dtype=jnp.int32)
    src_p = jnp.concatenate([src, pad_ar % N])
    dst_p = jnp.concatenate([dst, N + pad_ar % (NPAD - N)])
    src3 = src_p.reshape(NW, J, CHUNK)
    dst3 = dst_p.reshape(NW, J, CHUNK)
    zeros = jnp.zeros((ZCH, F), jnp.float32)

    zeros16 = jnp.zeros((ZCH, FD), jnp.float32)
    ones16 = jnp.ones((CHUNK, FD), jnp.float32)
    pdeg = _sc_degree(dst3, zeros16, ones16)
    dis, z0 = _tc_prep(pdeg, x)

    x2, z2 = _cheb_layer(x, z0, dis, W1, b1, gamma1, beta1, src3, dst3, zeros)
    y = _cheb_layer(x2, z2, dis, W2, b2, gamma2, beta2, src3, dst3, zeros,
                    res=x)
    return y


# speedup vs baseline: 1.0019x; 1.0014x over previous
import functools

import jax
import jax.numpy as jnp
from jax import lax
from jax.experimental import pallas as pl
from jax.experimental.pallas import tpu as pltpu
from jax.experimental.pallas import tpu_sc as plsc

N = 10000
F = 128
E = 320000

NC = 2
NS = 16
NW = NC * NS

CHUNK = 128
NBUF = 2
IDEP = 4
J = 80
E_PAD = NW * CHUNK * J
NPAD = 10240
ROWS_PER_TILE = NPAD // NS
ZCH = 128

_mesh = plsc.VectorSubcoreMesh(core_axis_name="c", subcore_axis_name="s")


@jax.jit
def _sc_aggregate(z, src3, dst3, zeros):
    @functools.partial(
        pl.kernel,
        out_type=jax.ShapeDtypeStruct((NC, NPAD, F), jnp.float32),
        mesh=_mesh,
        scratch_types=[
            pltpu.VMEM((IDEP, CHUNK), jnp.int32),
            pltpu.VMEM((IDEP, CHUNK), jnp.int32),
            pltpu.VMEM((NBUF, CHUNK, F), jnp.float32),
            pltpu.VMEM_SHARED((NPAD, F), jnp.float32),
        ] + [pltpu.SemaphoreType.DMA] * (NBUF + 2 * IDEP),
    )
    def agg_kernel(z_hbm, src_hbm, dst_hbm, zeros_hbm, out_hbm,
                   src_r, dst_r, rows_v, agg_sh, *sems):
        cid = lax.axis_index("c")
        sid = lax.axis_index("s")
        wid = cid * NS + sid
        gsem = sems[:NBUF]
        isem_s = sems[NBUF:NBUF + IDEP]
        isem_d = sems[NBUF + IDEP:]

        def pref_idx(q, j):
            pltpu.async_copy(src_hbm.at[wid, j], src_r.at[q], isem_s[q])
            pltpu.async_copy(dst_hbm.at[wid, j], dst_r.at[q], isem_d[q])

        def gather(b, q, j):
            pltpu.make_async_copy(
                src_hbm.at[wid, j], src_r.at[q], isem_s[q]).wait()
            pltpu.async_copy(z_hbm.at[src_r.at[q]], rows_v.at[b], gsem[b])

        def scatter(b, q, j):
            pltpu.make_async_copy(
                z_hbm.at[src_r.at[q]], rows_v.at[b], gsem[b]).wait()
            pltpu.make_async_copy(
                dst_hbm.at[wid, j], dst_r.at[q], isem_d[q]).wait()
            pltpu.sync_copy(rows_v.at[b], agg_sh.at[dst_r.at[q]], add=True)

        for q in range(IDEP):
            pref_idx(q, q)

        row0 = sid * ROWS_PER_TILE
        @pl.loop(0, ROWS_PER_TILE, step=ZCH)
        def _(r):
            pltpu.sync_copy(zeros_hbm, agg_sh.at[pl.ds(row0 + r, ZCH)])

        plsc.subcore_barrier()

        for b in range(NBUF):
            gather(b, b, b)

        @pl.loop(0, J - IDEP, step=IDEP)
        def _(j):
            for b in range(IDEP):
                jb = j + b
                scatter(b % NBUF, b, jb)
                pref_idx(b, jb + IDEP)
                gather(b % NBUF, (b + NBUF) % IDEP, jb + NBUF)

        for b in range(IDEP):
            jb = J - IDEP + b
            scatter(b % NBUF, b, jb)
            if b < IDEP - NBUF:
                gather(b % NBUF, (b + NBUF) % IDEP, jb + NBUF)

        plsc.subcore_barrier()

        @pl.loop(0, ROWS_PER_TILE, step=ZCH)
        def _(r):
            pltpu.sync_copy(agg_sh.at[pl.ds(row0 + r, ZCH)],
                            out_hbm.at[cid, pl.ds(row0 + r, ZCH)])

    return agg_kernel(z, src3, dst3, zeros)


FD = 128

@jax.jit
def _sc_degree(dst3, zeros16, ones16):
    @functools.partial(
        pl.kernel,
        out_type=jax.ShapeDtypeStruct((NC, NPAD, FD), jnp.float32),
        mesh=_mesh,
        scratch_types=[
            pltpu.VMEM((IDEP, CHUNK), jnp.int32),
            pltpu.VMEM((CHUNK, FD), jnp.float32),
            pltpu.VMEM_SHARED((NPAD, FD), jnp.float32),
        ] + [pltpu.SemaphoreType.DMA] * IDEP,
    )
    def deg_kernel(dst_hbm, zeros_hbm, ones_hbm, out_hbm,
                   dst_r, ones_v, deg_sh, *isem):
        cid = lax.axis_index("c")
        sid = lax.axis_index("s")
        wid = cid * NS + sid

        for q in range(IDEP):
            pltpu.async_copy(dst_hbm.at[wid, q], dst_r.at[q], isem[q])
        pltpu.sync_copy(ones_hbm, ones_v)

        row0 = sid * ROWS_PER_TILE
        @pl.loop(0, ROWS_PER_TILE, step=ZCH)
        def _(r):
            pltpu.sync_copy(zeros_hbm, deg_sh.at[pl.ds(row0 + r, ZCH)])

        plsc.subcore_barrier()

        @pl.loop(0, J - IDEP, step=IDEP)
        def _(j):
            for b in range(IDEP):
                pltpu.make_async_copy(
                    dst_hbm.at[wid, j + b], dst_r.at[b], isem[b]).wait()
                pltpu.sync_copy(ones_v, deg_sh.at[dst_r.at[b]], add=True)
                pltpu.async_copy(
                    dst_hbm.at[wid, j + b + IDEP], dst_r.at[b], isem[b])

        for b in range(IDEP):
            jb = J - IDEP + b
            pltpu.make_async_copy(
                dst_hbm.at[wid, jb], dst_r.at[b], isem[b]).wait()
            pltpu.sync_copy(ones_v, deg_sh.at[dst_r.at[b]], add=True)

        plsc.subcore_barrier()

        @pl.loop(0, ROWS_PER_TILE, step=ZCH)
        def _(r):
            pltpu.sync_copy(deg_sh.at[pl.ds(row0 + r, ZCH)],
                            out_hbm.at[cid, pl.ds(row0 + r, ZCH)])

    return deg_kernel(dst3, zeros16, ones16)


BN = 400
NBLK = N // BN

_row = pl.BlockSpec((BN, F), lambda i: (i, 0))
_prt = pl.BlockSpec((NC, BN, F), lambda i: (0, i, 0))
_dis = pl.BlockSpec((BN, 1), lambda i: (i, 0))
_wsp = pl.BlockSpec((F, F), lambda i: (0, 0))
_vec = pl.BlockSpec((1, F), lambda i: (0, 0))

_rowF = jax.ShapeDtypeStruct((N, F), jnp.float32)
_disF = jax.ShapeDtypeStruct((NPAD, 1), jnp.float32)


def _tc_prep(pdeg, x):
    def body(p_ref, x_ref, dis_ref, z_ref):
        deg = p_ref[0, :, 0] + p_ref[1, :, 0]
        dis = jnp.where(deg > 0.0, lax.rsqrt(jnp.maximum(deg, 1.0)), 0.0)
        dis_ref[...] = dis[:, None]
        z_ref[...] = x_ref[...] * dis[:N, None]

    return pl.pallas_call(
        body,
        grid=(1,),
        in_specs=[pl.BlockSpec((NC, NPAD, FD), lambda i: (0, 0, 0)),
                  pl.BlockSpec((N, F), lambda i: (0, 0))],
        out_specs=[pl.BlockSpec((NPAD, 1), lambda i: (0, 0)),
                   pl.BlockSpec((N, F), lambda i: (0, 0))],
        out_shape=[_disF, _rowF],
    )(pdeg, x)


def _tc_step1(P, dis, x, w0, w1):
    def body(p_ref, dis_ref, x_ref, w0_ref, w1_ref, out_ref, t_ref, z_ref):
        d = dis_ref[...]
        s = -(p_ref[0] + p_ref[1]) * d
        t_ref[...] = s
        z_ref[...] = s * d
        out_ref[...] = (
            jnp.dot(x_ref[...], w0_ref[...], preferred_element_type=jnp.float32)
            + jnp.dot(s, w1_ref[...], preferred_element_type=jnp.float32))

    return pl.pallas_call(
        body,
        grid=(NBLK,),
        in_specs=[_prt, _dis, _row, _wsp, _wsp],
        out_specs=[_row, _row, _row],
        out_shape=[_rowF, _rowF, _rowF],
    )(P, dis, x, w0, w1)


def _tc_step_mid(P, dis, tpp, out_in, wk):
    def body(p_ref, dis_ref, tpp_ref, o_ref, wk_ref, out_ref, t_ref, z_ref):
        d = dis_ref[...]
        t = -2.0 * (p_ref[0] + p_ref[1]) * d - tpp_ref[...]
        t_ref[...] = t
        z_ref[...] = t * d
        out_ref[...] = o_ref[...] + jnp.dot(
            t, wk_ref[...], preferred_element_type=jnp.float32)

    return pl.pallas_call(
        body,
        grid=(NBLK,),
        in_specs=[_prt, _dis, _row, _row, _wsp],
        out_specs=[_row, _row, _row],
        out_shape=[_rowF, _rowF, _rowF],
    )(P, dis, tpp, out_in, wk)


def _tc_step_final(P, dis, tpp, out_in, wk, b, gamma, beta, res=None):
    last = res is not None

    def body(*refs):
        if last:
            (p_ref, dis_ref, tpp_ref, o_ref, wk_ref, b_ref, g_ref, be_ref,
             r_ref, y_ref) = refs
        else:
            (p_ref, dis_ref, tpp_ref, o_ref, wk_ref, b_ref, g_ref, be_ref,
             xn_ref, z_ref) = refs
        d = dis_ref[...]
        t = -2.0 * (p_ref[0] + p_ref[1]) * d - tpp_ref[...]
        o = o_ref[...] + jnp.dot(
            t, wk_ref[...], preferred_element_type=jnp.float32) + b_ref[...]
        h = jnp.maximum(o, 0.0)
        mu = jnp.mean(h, axis=-1, keepdims=True)
        var = jnp.mean((h - mu) ** 2, axis=-1, keepdims=True)
        hn = (h - mu) / jnp.sqrt(var + 1e-6) * g_ref[...] + be_ref[...]
        if last:
            y_ref[...] = hn + r_ref[...]
        else:
            xn_ref[...] = hn
            z_ref[...] = hn * d

    in_specs = [_prt, _dis, _row, _row, _wsp, _vec, _vec, _vec]
    args = [P, dis, tpp, out_in, wk, b, gamma, beta]
    if last:
        in_specs.append(_row)
        args.append(res)
        out_specs, out_shape = [_row], [_rowF]
    else:
        out_specs, out_shape = [_row, _row], [_rowF, _rowF]

    result = pl.pallas_call(
        body,
        grid=(NBLK,),
        in_specs=in_specs,
        out_specs=out_specs,
        out_shape=out_shape,
    )(*args)
    return result[0] if last else result


def _cheb_layer(x, z, dis, W, b, gamma, beta, src3, dst3, zeros, res=None):
    K = W.shape[0]
    P = _sc_aggregate(z, src3, dst3, zeros)
    out, t1, z = _tc_step1(P, dis, x, W[0], W[1])
    tprev2, tprev1 = x, t1
    for k in range(2, K - 1):
        P = _sc_aggregate(z, src3, dst3, zeros)
        out, tk, z = _tc_step_mid(P, dis, tprev2, out, W[k])
        tprev2, tprev1 = tprev1, tk
    P = _sc_aggregate(z, src3, dst3, zeros)
    b2 = b.reshape(1, F)
    g2 = gamma.reshape(1, F)
    be2 = beta.reshape(1, F)
    return _tc_step_final(P, dis, tprev2, out, W[K - 1], b2, g2, be2, res=res)


def kernel(x, edge_index, W1, b1, gamma1, beta1, W2, b2, gamma2, beta2):
    src = edge_index[0]
    dst = edge_index[1]
    pad = E_PAD - E
    pad_ar = jnp.arange(pad, dtype=jnp.int32)
    src_p = jnp.concatenate([src, pad_ar % N])
    dst_p = jnp.concatenate([dst, N + pad_ar % (NPAD - N)])
    src3 = src_p.reshape(NW, J, CHUNK)
    dst3 = dst_p.reshape(NW, J, CHUNK)
    zeros = jnp.zeros((ZCH, F), jnp.float32)

    zeros16 = jnp.zeros((ZCH, FD), jnp.float32)
    ones16 = jnp.ones((CHUNK, FD), jnp.float32)
    pdeg = _sc_degree(dst3, zeros16, ones16)
    dis, z0 = _tc_prep(pdeg, x)

    x2, z2 = _cheb_layer(x, z0, dis, W1, b1, gamma1, beta1, src3, dst3, zeros)
    y = _cheb_layer(x2, z2, dis, W2, b2, gamma2, beta2, src3, dst3, zeros,
                    res=x)
    return y


# speedup vs baseline: 1.0787x; 1.0766x over previous
import functools

import jax
import jax.numpy as jnp
from jax import lax
from jax.experimental import pallas as pl
from jax.experimental.pallas import tpu as pltpu
from jax.experimental.pallas import tpu_sc as plsc

N = 10000
F = 128
E = 320000

NC = 2
NS = 16
NW = NC * NS

CHUNK = 128
NBUF = 2
IDEP = 4
J = 80
E_PAD = NW * CHUNK * J
NPAD = 10240
ROWS_PER_TILE = NPAD // NS
ZCH = 640

_mesh = plsc.VectorSubcoreMesh(core_axis_name="c", subcore_axis_name="s")


@jax.jit
def _sc_aggregate(z, src3, dst3, zeros):
    @functools.partial(
        pl.kernel,
        out_type=jax.ShapeDtypeStruct((NC, NPAD, F), jnp.float32),
        mesh=_mesh,
        scratch_types=[
            pltpu.VMEM((IDEP, CHUNK), jnp.int32),
            pltpu.VMEM((IDEP, CHUNK), jnp.int32),
            pltpu.VMEM((NBUF, CHUNK, F), jnp.float32),
            pltpu.VMEM_SHARED((NPAD, F), jnp.float32),
        ] + [pltpu.SemaphoreType.DMA] * (NBUF + 2 * IDEP),
    )
    def agg_kernel(z_hbm, src_hbm, dst_hbm, zeros_hbm, out_hbm,
                   src_r, dst_r, rows_v, agg_sh, *sems):
        cid = lax.axis_index("c")
        sid = lax.axis_index("s")
        wid = cid * NS + sid
        gsem = sems[:NBUF]
        isem_s = sems[NBUF:NBUF + IDEP]
        isem_d = sems[NBUF + IDEP:]

        def pref_idx(q, j):
            pltpu.async_copy(src_hbm.at[wid, j], src_r.at[q], isem_s[q])
            pltpu.async_copy(dst_hbm.at[wid, j], dst_r.at[q], isem_d[q])

        def gather(b, q, j):
            pltpu.make_async_copy(
                src_hbm.at[wid, j], src_r.at[q], isem_s[q]).wait()
            pltpu.async_copy(z_hbm.at[src_r.at[q]], rows_v.at[b], gsem[b])

        def scatter(b, q, j):
            pltpu.make_async_copy(
                z_hbm.at[src_r.at[q]], rows_v.at[b], gsem[b]).wait()
            pltpu.make_async_copy(
                dst_hbm.at[wid, j], dst_r.at[q], isem_d[q]).wait()
            pltpu.sync_copy(rows_v.at[b], agg_sh.at[dst_r.at[q]], add=True)

        for q in range(IDEP):
            pref_idx(q, q)

        row0 = sid * ROWS_PER_TILE
        @pl.loop(0, ROWS_PER_TILE, step=ZCH)
        def _(r):
            pltpu.sync_copy(zeros_hbm, agg_sh.at[pl.ds(row0 + r, ZCH)])

        plsc.subcore_barrier()

        for b in range(NBUF):
            gather(b, b, b)

        @pl.loop(0, J - IDEP, step=IDEP)
        def _(j):
            for b in range(IDEP):
                jb = j + b
                scatter(b % NBUF, b, jb)
                pref_idx(b, jb + IDEP)
                gather(b % NBUF, (b + NBUF) % IDEP, jb + NBUF)

        for b in range(IDEP):
            jb = J - IDEP + b
            scatter(b % NBUF, b, jb)
            if b < IDEP - NBUF:
                gather(b % NBUF, (b + NBUF) % IDEP, jb + NBUF)

        plsc.subcore_barrier()

        @pl.loop(0, ROWS_PER_TILE, step=ZCH)
        def _(r):
            pltpu.sync_copy(agg_sh.at[pl.ds(row0 + r, ZCH)],
                            out_hbm.at[cid, pl.ds(row0 + r, ZCH)])

    return agg_kernel(z, src3, dst3, zeros)


FD = 128

@jax.jit
def _sc_degree(dst3, zeros16, ones16):
    @functools.partial(
        pl.kernel,
        out_type=jax.ShapeDtypeStruct((NC, NPAD, FD), jnp.float32),
        mesh=_mesh,
        scratch_types=[
            pltpu.VMEM((IDEP, CHUNK), jnp.int32),
            pltpu.VMEM((CHUNK, FD), jnp.float32),
            pltpu.VMEM_SHARED((NPAD, FD), jnp.float32),
        ] + [pltpu.SemaphoreType.DMA] * IDEP,
    )
    def deg_kernel(dst_hbm, zeros_hbm, ones_hbm, out_hbm,
                   dst_r, ones_v, deg_sh, *isem):
        cid = lax.axis_index("c")
        sid = lax.axis_index("s")
        wid = cid * NS + sid

        for q in range(IDEP):
            pltpu.async_copy(dst_hbm.at[wid, q], dst_r.at[q], isem[q])
        pltpu.sync_copy(ones_hbm, ones_v)

        row0 = sid * ROWS_PER_TILE
        @pl.loop(0, ROWS_PER_TILE, step=ZCH)
        def _(r):
            pltpu.sync_copy(zeros_hbm, deg_sh.at[pl.ds(row0 + r, ZCH)])

        plsc.subcore_barrier()

        @pl.loop(0, J - IDEP, step=IDEP)
        def _(j):
            for b in range(IDEP):
                pltpu.make_async_copy(
                    dst_hbm.at[wid, j + b], dst_r.at[b], isem[b]).wait()
                pltpu.sync_copy(ones_v, deg_sh.at[dst_r.at[b]], add=True)
                pltpu.async_copy(
                    dst_hbm.at[wid, j + b + IDEP], dst_r.at[b], isem[b])

        for b in range(IDEP):
            jb = J - IDEP + b
            pltpu.make_async_copy(
                dst_hbm.at[wid, jb], dst_r.at[b], isem[b]).wait()
            pltpu.sync_copy(ones_v, deg_sh.at[dst_r.at[b]], add=True)

        plsc.subcore_barrier()

        @pl.loop(0, ROWS_PER_TILE, step=ZCH)
        def _(r):
            pltpu.sync_copy(deg_sh.at[pl.ds(row0 + r, ZCH)],
                            out_hbm.at[cid, pl.ds(row0 + r, ZCH)])

    return deg_kernel(dst3, zeros16, ones16)


BN = 400
NBLK = N // BN

_row = pl.BlockSpec((BN, F), lambda i: (i, 0))
_prt = pl.BlockSpec((NC, BN, F), lambda i: (0, i, 0))
_dis = pl.BlockSpec((BN, 1), lambda i: (i, 0))
_wsp = pl.BlockSpec((F, F), lambda i: (0, 0))
_vec = pl.BlockSpec((1, F), lambda i: (0, 0))

_rowF = jax.ShapeDtypeStruct((N, F), jnp.float32)
_disF = jax.ShapeDtypeStruct((NPAD, 1), jnp.float32)


def _tc_prep(pdeg, x):
    def body(p_ref, x_ref, dis_ref, z_ref):
        deg = p_ref[0, :, 0] + p_ref[1, :, 0]
        dis = jnp.where(deg > 0.0, lax.rsqrt(jnp.maximum(deg, 1.0)), 0.0)
        dis_ref[...] = dis[:, None]
        z_ref[...] = x_ref[...] * dis[:N, None]

    return pl.pallas_call(
        body,
        grid=(1,),
        in_specs=[pl.BlockSpec((NC, NPAD, FD), lambda i: (0, 0, 0)),
                  pl.BlockSpec((N, F), lambda i: (0, 0))],
        out_specs=[pl.BlockSpec((NPAD, 1), lambda i: (0, 0)),
                   pl.BlockSpec((N, F), lambda i: (0, 0))],
        out_shape=[_disF, _rowF],
    )(pdeg, x)


def _tc_step1(P, dis, x, w0, w1):
    def body(p_ref, dis_ref, x_ref, w0_ref, w1_ref, out_ref, t_ref, z_ref):
        d = dis_ref[...]
        s = -(p_ref[0] + p_ref[1]) * d
        t_ref[...] = s
        z_ref[...] = s * d
        out_ref[...] = (
            jnp.dot(x_ref[...], w0_ref[...], preferred_element_type=jnp.float32)
            + jnp.dot(s, w1_ref[...], preferred_element_type=jnp.float32))

    return pl.pallas_call(
        body,
        grid=(NBLK,),
        in_specs=[_prt, _dis, _row, _wsp, _wsp],
        out_specs=[_row, _row, _row],
        out_shape=[_rowF, _rowF, _rowF],
    )(P, dis, x, w0, w1)


def _tc_step_mid(P, dis, tpp, out_in, wk):
    def body(p_ref, dis_ref, tpp_ref, o_ref, wk_ref, out_ref, t_ref, z_ref):
        d = dis_ref[...]
        t = -2.0 * (p_ref[0] + p_ref[1]) * d - tpp_ref[...]
        t_ref[...] = t
        z_ref[...] = t * d
        out_ref[...] = o_ref[...] + jnp.dot(
            t, wk_ref[...], preferred_element_type=jnp.float32)

    return pl.pallas_call(
        body,
        grid=(NBLK,),
        in_specs=[_prt, _dis, _row, _row, _wsp],
        out_specs=[_row, _row, _row],
        out_shape=[_rowF, _rowF, _rowF],
    )(P, dis, tpp, out_in, wk)


def _tc_step_final(P, dis, tpp, out_in, wk, b, gamma, beta, res=None):
    last = res is not None

    def body(*refs):
        if last:
            (p_ref, dis_ref, tpp_ref, o_ref, wk_ref, b_ref, g_ref, be_ref,
             r_ref, y_ref) = refs
        else:
            (p_ref, dis_ref, tpp_ref, o_ref, wk_ref, b_ref, g_ref, be_ref,
             xn_ref, z_ref) = refs
        d = dis_ref[...]
        t = -2.0 * (p_ref[0] + p_ref[1]) * d - tpp_ref[...]
        o = o_ref[...] + jnp.dot(
            t, wk_ref[...], preferred_element_type=jnp.float32) + b_ref[...]
        h = jnp.maximum(o, 0.0)
        mu = jnp.mean(h, axis=-1, keepdims=True)
        var = jnp.mean((h - mu) ** 2, axis=-1, keepdims=True)
        hn = (h - mu) / jnp.sqrt(var + 1e-6) * g_ref[...] + be_ref[...]
        if last:
            y_ref[...] = hn + r_ref[...]
        else:
            xn_ref[...] = hn
            z_ref[...] = hn * d

    in_specs = [_prt, _dis, _row, _row, _wsp, _vec, _vec, _vec]
    args = [P, dis, tpp, out_in, wk, b, gamma, beta]
    if last:
        in_specs.append(_row)
        args.append(res)
        out_specs, out_shape = [_row], [_rowF]
    else:
        out_specs, out_shape = [_row, _row], [_rowF, _rowF]

    result = pl.pallas_call(
        body,
        grid=(NBLK,),
        in_specs=in_specs,
        out_specs=out_specs,
        out_shape=out_shape,
    )(*args)
    return result[0] if last else result


def _cheb_layer(x, z, dis, W, b, gamma, beta, src3, dst3, zeros, res=None):
    K = W.shape[0]
    P = _sc_aggregate(z, src3, dst3, zeros)
    out, t1, z = _tc_step1(P, dis, x, W[0], W[1])
    tprev2, tprev1 = x, t1
    for k in range(2, K - 1):
        P = _sc_aggregate(z, src3, dst3, zeros)
        out, tk, z = _tc_step_mid(P, dis, tprev2, out, W[k])
        tprev2, tprev1 = tprev1, tk
    P = _sc_aggregate(z, src3, dst3, zeros)
    b2 = b.reshape(1, F)
    g2 = gamma.reshape(1, F)
    be2 = beta.reshape(1, F)
    return _tc_step_final(P, dis, tprev2, out, W[K - 1], b2, g2, be2, res=res)


def kernel(x, edge_index, W1, b1, gamma1, beta1, W2, b2, gamma2, beta2):
    src = edge_index[0]
    dst = edge_index[1]
    pad = E_PAD - E
    pad_ar = jnp.arange(pad, dtype=jnp.int32)
    src_p = jnp.concatenate([src, pad_ar % N])
    dst_p = jnp.concatenate([dst, N + pad_ar % (NPAD - N)])
    src3 = src_p.reshape(NW, J, CHUNK)
    dst3 = dst_p.reshape(NW, J, CHUNK)
    zeros = jnp.zeros((ZCH, F), jnp.float32)

    zeros16 = jnp.zeros((ZCH, FD), jnp.float32)
    ones16 = jnp.ones((CHUNK, FD), jnp.float32)
    pdeg = _sc_degree(dst3, zeros16, ones16)
    dis, z0 = _tc_prep(pdeg, x)

    x2, z2 = _cheb_layer(x, z0, dis, W1, b1, gamma1, beta1, src3, dst3, zeros)
    y = _cheb_layer(x2, z2, dis, W2, b2, gamma2, beta2, src3, dst3, zeros,
                    res=x)
    return y


# speedup vs baseline: 1.0843x; 1.0052x over previous
import functools

import jax
import jax.numpy as jnp
from jax import lax
from jax.experimental import pallas as pl
from jax.experimental.pallas import tpu as pltpu
from jax.experimental.pallas import tpu_sc as plsc

N = 10000
F = 128
E = 320000

NC = 2
NS = 16
NW = NC * NS

CHUNK = 128
NBUF = 2
IDEP = 4
J = 80
E_PAD = NW * CHUNK * J
NPAD = 10240
ROWS_PER_TILE = NPAD // NS
ZCH = 640

_mesh = plsc.VectorSubcoreMesh(core_axis_name="c", subcore_axis_name="s")


@jax.jit
def _sc_aggregate(z, src3, dst3, zeros):
    @functools.partial(
        pl.kernel,
        out_type=jax.ShapeDtypeStruct((NC, NPAD, F), jnp.float32),
        mesh=_mesh,
        scratch_types=[
            pltpu.VMEM((IDEP, CHUNK), jnp.int32),
            pltpu.VMEM((IDEP, CHUNK), jnp.int32),
            pltpu.VMEM((NBUF, CHUNK, F), jnp.float32),
            pltpu.VMEM_SHARED((NPAD, F), jnp.float32),
        ] + [pltpu.SemaphoreType.DMA] * (NBUF + 2 * IDEP + 1),
    )
    def agg_kernel(z_hbm, src_hbm, dst_hbm, zeros_hbm, out_hbm,
                   src_r, dst_r, rows_v, agg_sh, *sems):
        cid = lax.axis_index("c")
        sid = lax.axis_index("s")
        wid = cid * NS + sid
        gsem = sems[:NBUF]
        isem_s = sems[NBUF:NBUF + IDEP]
        isem_d = sems[NBUF + IDEP:NBUF + 2 * IDEP]
        zsem = sems[NBUF + 2 * IDEP]

        def pref_idx(q, j):
            pltpu.async_copy(src_hbm.at[wid, j], src_r.at[q], isem_s[q])
            pltpu.async_copy(dst_hbm.at[wid, j], dst_r.at[q], isem_d[q])

        def gather(b, q, j):
            pltpu.make_async_copy(
                src_hbm.at[wid, j], src_r.at[q], isem_s[q]).wait()
            pltpu.async_copy(z_hbm.at[src_r.at[q]], rows_v.at[b], gsem[b])

        def scatter(b, q, j):
            pltpu.make_async_copy(
                z_hbm.at[src_r.at[q]], rows_v.at[b], gsem[b]).wait()
            pltpu.make_async_copy(
                dst_hbm.at[wid, j], dst_r.at[q], isem_d[q]).wait()
            pltpu.sync_copy(rows_v.at[b], agg_sh.at[dst_r.at[q]], add=True)

        for q in range(IDEP):
            pref_idx(q, q)

        row0 = sid * ROWS_PER_TILE
        pltpu.async_copy(zeros_hbm, agg_sh.at[pl.ds(row0, ZCH)], zsem)

        for b in range(NBUF):
            gather(b, b, b)

        pltpu.make_async_copy(
            zeros_hbm, agg_sh.at[pl.ds(row0, ZCH)], zsem).wait()
        plsc.subcore_barrier()

        @pl.loop(0, J - IDEP, step=IDEP)
        def _(j):
            for b in range(IDEP):
                jb = j + b
                scatter(b % NBUF, b, jb)
                pref_idx(b, jb + IDEP)
                gather(b % NBUF, (b + NBUF) % IDEP, jb + NBUF)

        for b in range(IDEP):
            jb = J - IDEP + b
            scatter(b % NBUF, b, jb)
            if b < IDEP - NBUF:
                gather(b % NBUF, (b + NBUF) % IDEP, jb + NBUF)

        plsc.subcore_barrier()

        @pl.loop(0, ROWS_PER_TILE, step=ZCH)
        def _(r):
            pltpu.sync_copy(agg_sh.at[pl.ds(row0 + r, ZCH)],
                            out_hbm.at[cid, pl.ds(row0 + r, ZCH)])

    return agg_kernel(z, src3, dst3, zeros)


FD = 128

@jax.jit
def _sc_degree(dst3, zeros16, ones16):
    @functools.partial(
        pl.kernel,
        out_type=jax.ShapeDtypeStruct((NC, NPAD, FD), jnp.float32),
        mesh=_mesh,
        scratch_types=[
            pltpu.VMEM((IDEP, CHUNK), jnp.int32),
            pltpu.VMEM((CHUNK, FD), jnp.float32),
            pltpu.VMEM_SHARED((NPAD, FD), jnp.float32),
        ] + [pltpu.SemaphoreType.DMA] * IDEP,
    )
    def deg_kernel(dst_hbm, zeros_hbm, ones_hbm, out_hbm,
                   dst_r, ones_v, deg_sh, *isem):
        cid = lax.axis_index("c")
        sid = lax.axis_index("s")
        wid = cid * NS + sid

        for q in range(IDEP):
            pltpu.async_copy(dst_hbm.at[wid, q], dst_r.at[q], isem[q])
        pltpu.sync_copy(ones_hbm, ones_v)

        row0 = sid * ROWS_PER_TILE
        @pl.loop(0, ROWS_PER_TILE, step=ZCH)
        def _(r):
            pltpu.sync_copy(zeros_hbm, deg_sh.at[pl.ds(row0 + r, ZCH)])

        plsc.subcore_barrier()

        @pl.loop(0, J - IDEP, step=IDEP)
        def _(j):
            for b in range(IDEP):
                pltpu.make_async_copy(
                    dst_hbm.at[wid, j + b], dst_r.at[b], isem[b]).wait()
                pltpu.sync_copy(ones_v, deg_sh.at[dst_r.at[b]], add=True)
                pltpu.async_copy(
                    dst_hbm.at[wid, j + b + IDEP], dst_r.at[b], isem[b])

        for b in range(IDEP):
            jb = J - IDEP + b
            pltpu.make_async_copy(
                dst_hbm.at[wid, jb], dst_r.at[b], isem[b]).wait()
            pltpu.sync_copy(ones_v, deg_sh.at[dst_r.at[b]], add=True)

        plsc.subcore_barrier()

        @pl.loop(0, ROWS_PER_TILE, step=ZCH)
        def _(r):
            pltpu.sync_copy(deg_sh.at[pl.ds(row0 + r, ZCH)],
                            out_hbm.at[cid, pl.ds(row0 + r, ZCH)])

    return deg_kernel(dst3, zeros16, ones16)


BN = 400
NBLK = N // BN

_row = pl.BlockSpec((BN, F), lambda i: (i, 0))
_prt = pl.BlockSpec((NC, BN, F), lambda i: (0, i, 0))
_dis = pl.BlockSpec((BN, 1), lambda i: (i, 0))
_wsp = pl.BlockSpec((F, F), lambda i: (0, 0))
_vec = pl.BlockSpec((1, F), lambda i: (0, 0))

_rowF = jax.ShapeDtypeStruct((N, F), jnp.float32)
_disF = jax.ShapeDtypeStruct((NPAD, 1), jnp.float32)


def _tc_prep(pdeg, x):
    def body(p_ref, x_ref, dis_ref, z_ref):
        deg = p_ref[0, :, 0] + p_ref[1, :, 0]
        dis = jnp.where(deg > 0.0, lax.rsqrt(jnp.maximum(deg, 1.0)), 0.0)
        dis_ref[...] = dis[:, None]
        z_ref[...] = x_ref[...] * dis[:N, None]

    return pl.pallas_call(
        body,
        grid=(1,),
        in_specs=[pl.BlockSpec((NC, NPAD, FD), lambda i: (0, 0, 0)),
                  pl.BlockSpec((N, F), lambda i: (0, 0))],
        out_specs=[pl.BlockSpec((NPAD, 1), lambda i: (0, 0)),
                   pl.BlockSpec((N, F), lambda i: (0, 0))],
        out_shape=[_disF, _rowF],
    )(pdeg, x)


def _tc_step1(P, dis, x, w0, w1):
    def body(p_ref, dis_ref, x_ref, w0_ref, w1_ref, out_ref, t_ref, z_ref):
        d = dis_ref[...]
        s = -(p_ref[0] + p_ref[1]) * d
        t_ref[...] = s
        z_ref[...] = s * d
        out_ref[...] = (
            jnp.dot(x_ref[...], w0_ref[...], preferred_element_type=jnp.float32)
            + jnp.dot(s, w1_ref[...], preferred_element_type=jnp.float32))

    return pl.pallas_call(
        body,
        grid=(NBLK,),
        in_specs=[_prt, _dis, _row, _wsp, _wsp],
        out_specs=[_row, _row, _row],
        out_shape=[_rowF, _rowF, _rowF],
    )(P, dis, x, w0, w1)


def _tc_step_mid(P, dis, tpp, out_in, wk):
    def body(p_ref, dis_ref, tpp_ref, o_ref, wk_ref, out_ref, t_ref, z_ref):
        d = dis_ref[...]
        t = -2.0 * (p_ref[0] + p_ref[1]) * d - tpp_ref[...]
        t_ref[...] = t
        z_ref[...] = t * d
        out_ref[...] = o_ref[...] + jnp.dot(
            t, wk_ref[...], preferred_element_type=jnp.float32)

    return pl.pallas_call(
        body,
        grid=(NBLK,),
        in_specs=[_prt, _dis, _row, _row, _wsp],
        out_specs=[_row, _row, _row],
        out_shape=[_rowF, _rowF, _rowF],
    )(P, dis, tpp, out_in, wk)


def _tc_step_final(P, dis, tpp, out_in, wk, b, gamma, beta, res=None):
    last = res is not None

    def body(*refs):
        if last:
            (p_ref, dis_ref, tpp_ref, o_ref, wk_ref, b_ref, g_ref, be_ref,
             r_ref, y_ref) = refs
        else:
            (p_ref, dis_ref, tpp_ref, o_ref, wk_ref, b_ref, g_ref, be_ref,
             xn_ref, z_ref) = refs
        d = dis_ref[...]
        t = -2.0 * (p_ref[0] + p_ref[1]) * d - tpp_ref[...]
        o = o_ref[...] + jnp.dot(
            t, wk_ref[...], preferred_element_type=jnp.float32) + b_ref[...]
        h = jnp.maximum(o, 0.0)
        mu = jnp.mean(h, axis=-1, keepdims=True)
        var = jnp.mean((h - mu) ** 2, axis=-1, keepdims=True)
        hn = (h - mu) / jnp.sqrt(var + 1e-6) * g_ref[...] + be_ref[...]
        if last:
            y_ref[...] = hn + r_ref[...]
        else:
            xn_ref[...] = hn
            z_ref[...] = hn * d

    in_specs = [_prt, _dis, _row, _row, _wsp, _vec, _vec, _vec]
    args = [P, dis, tpp, out_in, wk, b, gamma, beta]
    if last:
        in_specs.append(_row)
        args.append(res)
        out_specs, out_shape = [_row], [_rowF]
    else:
        out_specs, out_shape = [_row, _row], [_rowF, _rowF]

    result = pl.pallas_call(
        body,
        grid=(NBLK,),
        in_specs=in_specs,
        out_specs=out_specs,
        out_shape=out_shape,
    )(*args)
    return result[0] if last else result


def _cheb_layer(x, z, dis, W, b, gamma, beta, src3, dst3, zeros, res=None):
    K = W.shape[0]
    P = _sc_aggregate(z, src3, dst3, zeros)
    out, t1, z = _tc_step1(P, dis, x, W[0], W[1])
    tprev2, tprev1 = x, t1
    for k in range(2, K - 1):
        P = _sc_aggregate(z, src3, dst3, zeros)
        out, tk, z = _tc_step_mid(P, dis, tprev2, out, W[k])
        tprev2, tprev1 = tprev1, tk
    P = _sc_aggregate(z, src3, dst3, zeros)
    b2 = b.reshape(1, F)
    g2 = gamma.reshape(1, F)
    be2 = beta.reshape(1, F)
    return _tc_step_final(P, dis, tprev2, out, W[K - 1], b2, g2, be2, res=res)


def kernel(x, edge_index, W1, b1, gamma1, beta1, W2, b2, gamma2, beta2):
    src = edge_index[0]
    dst = edge_index[1]
    pad = E_PAD - E
    pad_ar = jnp.arange(pad, dtype=jnp.int32)
    src_p = jnp.concatenate([src, pad_ar % N])
    dst_p = jnp.concatenate([dst, N + pad_ar % (NPAD - N)])
    src3 = src_p.reshape(NW, J, CHUNK)
    dst3 = dst_p.reshape(NW, J, CHUNK)
    zeros = jnp.zeros((ZCH, F), jnp.float32)

    zeros16 = jnp.zeros((ZCH, FD), jnp.float32)
    ones16 = jnp.ones((CHUNK, FD), jnp.float32)
    pdeg = _sc_degree(dst3, zeros16, ones16)
    dis, z0 = _tc_prep(pdeg, x)

    x2, z2 = _cheb_layer(x, z0, dis, W1, b1, gamma1, beta1, src3, dst3, zeros)
    y = _cheb_layer(x2, z2, dis, W2, b2, gamma2, beta2, src3, dst3, zeros,
                    res=x)
    return y
